# Initial kernel scaffold; baseline (speedup 1.0000x reference)
import jax
import jax.numpy as jnp
from jax.experimental import pallas as pl


def kernel(positions, input, edge_index, weights1, biases1, weights2, biases2):
    raise NotImplementedError("write your pallas kernel here")


# speedup vs baseline: 1.8447x; 1.8447x over previous
import functools

import jax
import jax.numpy as jnp
from jax import lax
from jax.experimental import pallas as pl
from jax.experimental.pallas import tpu as pltpu
from jax.experimental.pallas import tpu_sc as plsc

N_NODES = 10000
N_EDGES = 320000
NUM_GAUSSIANS = 128
NUM_FILTERS = 128
CUTOFF = 5.0
GAUSSIAN_WIDTH = CUTOFF / (NUM_GAUSSIANS - 1)

NC = 2
NS = 16
NW = NC * NS
EPW = N_EDGES // NW

ZCHUNK = 80
NCHUNK = N_NODES // ZCHUNK

B1 = 400
B3 = 80

def _d2_body(pos_hbm, src_hbm, dst_hbm, out_hbm, posv, srcv, dstv, d2v):
    wid = lax.axis_index("s") * NC + lax.axis_index("c")
    pltpu.sync_copy(pos_hbm, posv)

    def block(b, _):
        e0 = wid * EPW + b * B1
        pltpu.sync_copy(src_hbm.at[pl.ds(e0, B1)], srcv)
        pltpu.sync_copy(dst_hbm.at[pl.ds(e0, B1)], dstv)

        def inner(i, _):
            sl = pl.ds(i * 16, 16)
            si = srcv[sl] * 3
            di = dstv[sl] * 3
            dx = plsc.load_gather(posv, [si]) - plsc.load_gather(posv, [di])
            dy = plsc.load_gather(posv, [si + 1]) - plsc.load_gather(posv, [di + 1])
            dz = plsc.load_gather(posv, [si + 2]) - plsc.load_gather(posv, [di + 2])
            d2v[sl] = dx * dx + dy * dy + dz * dz
            return 0

        lax.fori_loop(0, B1 // 16, inner, 0)
        pltpu.sync_copy(d2v, out_hbm.at[pl.ds(e0, B1)])
        return 0

    lax.fori_loop(0, EPW // B1, block, 0)


EB = 1600


def _ssp(x):
    m = jnp.maximum(x, 0.0)
    return m + jnp.log(jnp.exp(x - m) + jnp.exp(-m)) - jnp.log(2.0)


def _filt_body(d2_ref, w1_ref, b1_ref, w2_ref, b2_ref, out_ref):
    d2 = d2_ref[...]
    r = jnp.sqrt(d2 + 1e-12)
    centers = (lax.broadcasted_iota(jnp.int32, (1, NUM_GAUSSIANS), 1)
               .astype(jnp.float32) * GAUSSIAN_WIDTH)
    t = r - centers
    inv2w2 = 1.0 / (2.0 * GAUSSIAN_WIDTH * GAUSSIAN_WIDTH)
    g = jnp.exp(-(t * t) * inv2w2)
    y = _ssp(jnp.dot(g, w1_ref[...],
                     preferred_element_type=jnp.float32,
                     precision=lax.Precision.HIGHEST) + b1_ref[...])
    w = _ssp(jnp.dot(y, w2_ref[...],
                     preferred_element_type=jnp.float32,
                     precision=lax.Precision.HIGHEST) + b2_ref[...])
    cut = jnp.where(r < CUTOFF, 0.5 * jnp.cos(jnp.pi * r / CUTOFF) + 0.5, 0.0)
    out_ref[...] = w * cut


_filt_call = pl.pallas_call(
    _filt_body,
    grid=(N_EDGES // EB,),
    in_specs=[
        pl.BlockSpec((EB, 1), lambda i: (i, 0)),
        pl.BlockSpec((NUM_GAUSSIANS, NUM_FILTERS), lambda i: (0, 0)),
        pl.BlockSpec((1, NUM_FILTERS), lambda i: (0, 0)),
        pl.BlockSpec((NUM_FILTERS, NUM_FILTERS), lambda i: (0, 0)),
        pl.BlockSpec((1, NUM_FILTERS), lambda i: (0, 0)),
    ],
    out_specs=pl.BlockSpec((EB, NUM_FILTERS), lambda i: (i, 0)),
    out_shape=jax.ShapeDtypeStruct((N_EDGES, NUM_FILTERS), jnp.float32),
)


def _scatter_body(in_hbm, src_hbm, dst_hbm, filt_hbm, out_hbm,
                  acc, srcv, dstv, filtv, inv, msgv, sem):
    c = lax.axis_index("c")
    s = lax.axis_index("s")
    wid = s * NC + c

    def zrow(i, _):
        for j in range(NUM_FILTERS // 16):
            msgv[i, pl.ds(j * 16, 16)] = jnp.zeros((16,), jnp.float32)
        return 0

    lax.fori_loop(0, ZCHUNK, zrow, 0)

    def zcopy(k, _):
        chunk = s + k * NS

        @pl.when(chunk < NCHUNK)
        def _():
            pltpu.sync_copy(msgv, acc.at[pl.ds(chunk * ZCHUNK, ZCHUNK)])

        return 0

    lax.fori_loop(0, (NCHUNK + NS - 1) // NS, zcopy, 0)
    plsc.subcore_barrier()

    def block(b, _):
        e0 = wid * EPW + b * B3
        pltpu.sync_copy(src_hbm.at[pl.ds(e0, B3)], srcv)
        pltpu.sync_copy(dst_hbm.at[pl.ds(e0, B3)], dstv)
        pltpu.sync_copy(filt_hbm.at[pl.ds(e0, B3)], filtv)
        pltpu.async_copy(in_hbm.at[srcv], inv, sem).wait()

        def row(i, _):
            for j in range(NUM_FILTERS // 16):
                sl = pl.ds(j * 16, 16)
                msgv[i, sl] = filtv[i, sl] * inv[i, sl]
            return 0

        lax.fori_loop(0, B3, row, 0)
        pltpu.sync_copy(msgv, acc.at[dstv], add=True)
        return 0

    lax.fori_loop(0, EPW // B3, block, 0)
    plsc.subcore_barrier()

    def dump(k, _):
        chunk = s + k * NS

        @pl.when(chunk < NCHUNK)
        def _():
            r0 = chunk * ZCHUNK
            pltpu.sync_copy(acc.at[pl.ds(r0, ZCHUNK)],
                            out_hbm.at[pl.ds(c * N_NODES + r0, ZCHUNK)])

        return 0

    lax.fori_loop(0, (NCHUNK + NS - 1) // NS, dump, 0)


def _sum_body(a_ref, b_ref, o_ref):
    o_ref[...] = a_ref[...] + b_ref[...]


_sum_call = pl.pallas_call(
    _sum_body,
    grid=(10,),
    in_specs=[
        pl.BlockSpec((N_NODES // 10, NUM_FILTERS), lambda i: (i, 0)),
        pl.BlockSpec((N_NODES // 10, NUM_FILTERS), lambda i: (i, 0)),
    ],
    out_specs=pl.BlockSpec((N_NODES // 10, NUM_FILTERS), lambda i: (i, 0)),
    out_shape=jax.ShapeDtypeStruct((N_NODES, NUM_FILTERS), jnp.float32),
)


@functools.lru_cache(maxsize=1)
def _sc_kernels():
    mesh = plsc.VectorSubcoreMesh(core_axis_name="c", subcore_axis_name="s",
                                  num_cores=NC, num_subcores=NS)
    d2_kernel = pl.kernel(
        _d2_body,
        out_type=jax.ShapeDtypeStruct((N_EDGES,), jnp.float32),
        mesh=mesh,
        compiler_params=pltpu.CompilerParams(needs_layout_passes=False),
        scratch_types=[
            pltpu.VMEM((3 * N_NODES,), jnp.float32),
            pltpu.VMEM((B1,), jnp.int32),
            pltpu.VMEM((B1,), jnp.int32),
            pltpu.VMEM((B1,), jnp.float32),
        ],
    )
    scatter_kernel = pl.kernel(
        _scatter_body,
        out_type=jax.ShapeDtypeStruct((NC * N_NODES, NUM_FILTERS),
                                      jnp.float32),
        mesh=mesh,
        compiler_params=pltpu.CompilerParams(needs_layout_passes=False),
        scratch_types=[
            pltpu.VMEM_SHARED((N_NODES, NUM_FILTERS), jnp.float32),
            pltpu.VMEM((B3,), jnp.int32),
            pltpu.VMEM((B3,), jnp.int32),
            pltpu.VMEM((B3, NUM_FILTERS), jnp.float32),
            pltpu.VMEM((B3, NUM_FILTERS), jnp.float32),
            pltpu.VMEM((B3, NUM_FILTERS), jnp.float32),
            pltpu.SemaphoreType.DMA,
        ],
    )
    return d2_kernel, scatter_kernel


def kernel(positions, input, edge_index, weights1, biases1, weights2, biases2):
    _d2_kernel, _scatter_kernel = _sc_kernels()
    pos_flat = positions.reshape(-1)
    src = edge_index[0]
    dst = edge_index[1]
    d2 = _d2_kernel(pos_flat, src, dst)
    filt = _filt_call(d2.reshape(N_EDGES, 1), weights1,
                      biases1.reshape(1, NUM_FILTERS), weights2,
                      biases2.reshape(1, NUM_FILTERS))
    parts = _scatter_kernel(input, src, dst, filt)
    return _sum_call(parts[:N_NODES], parts[N_NODES:])


# speedup vs baseline: 2.4153x; 1.3093x over previous
import functools

import jax
import jax.numpy as jnp
from jax import lax
from jax.experimental import pallas as pl
from jax.experimental.pallas import tpu as pltpu
from jax.experimental.pallas import tpu_sc as plsc

N_NODES = 10000
N_EDGES = 320000
NUM_GAUSSIANS = 128
NUM_FILTERS = 128
CUTOFF = 5.0
GAUSSIAN_WIDTH = CUTOFF / (NUM_GAUSSIANS - 1)

NC = 2
NS = 16
NW = NC * NS
EPW = N_EDGES // NW

ZCHUNK = 80
NCHUNK = N_NODES // ZCHUNK

B1 = 400
B3 = 80

def _d2_body(pos_hbm, src_hbm, dst_hbm, out_hbm, posv, srcv, dstv, d2v):
    wid = lax.axis_index("s") * NC + lax.axis_index("c")
    pltpu.sync_copy(pos_hbm, posv)

    def block(b, _):
        e0 = wid * EPW + b * B1
        pltpu.sync_copy(src_hbm.at[pl.ds(e0, B1)], srcv)
        pltpu.sync_copy(dst_hbm.at[pl.ds(e0, B1)], dstv)

        def inner(i, _):
            sl = pl.ds(i * 16, 16)
            si = srcv[sl] * 3
            di = dstv[sl] * 3
            dx = plsc.load_gather(posv, [si]) - plsc.load_gather(posv, [di])
            dy = plsc.load_gather(posv, [si + 1]) - plsc.load_gather(posv, [di + 1])
            dz = plsc.load_gather(posv, [si + 2]) - plsc.load_gather(posv, [di + 2])
            d2v[sl] = dx * dx + dy * dy + dz * dz
            return 0

        lax.fori_loop(0, B1 // 16, inner, 0)
        pltpu.sync_copy(d2v, out_hbm.at[pl.ds(e0, B1)])
        return 0

    lax.fori_loop(0, EPW // B1, block, 0)


EB = 1600


def _ssp(x):
    m = jnp.maximum(x, 0.0)
    return m + jnp.log(jnp.exp(x - m) + jnp.exp(-m)) - jnp.log(2.0)


def _filt_body(d2_ref, w1_ref, b1_ref, w2_ref, b2_ref, out_ref):
    d2 = d2_ref[...]
    r = jnp.sqrt(d2 + 1e-12)
    centers = (lax.broadcasted_iota(jnp.int32, (1, NUM_GAUSSIANS), 1)
               .astype(jnp.float32) * GAUSSIAN_WIDTH)
    t = r - centers
    inv2w2 = 1.0 / (2.0 * GAUSSIAN_WIDTH * GAUSSIAN_WIDTH)
    g = jnp.exp(-(t * t) * inv2w2)
    y = _ssp(jnp.dot(g, w1_ref[...],
                     preferred_element_type=jnp.float32,
                     precision=lax.Precision.HIGHEST) + b1_ref[...])
    w = _ssp(jnp.dot(y, w2_ref[...],
                     preferred_element_type=jnp.float32,
                     precision=lax.Precision.HIGHEST) + b2_ref[...])
    x = jnp.clip(0.5 - r * (1.0 / CUTOFF), -0.5, 0.5)
    z = x * x
    PI = 3.14159265358979
    p = x * (PI + z * (-PI**3 / 6.0 + z * (PI**5 / 120.0 + z * (
        -PI**7 / 5040.0 + z * (PI**9 / 362880.0)))))
    cut = jnp.where(r < CUTOFF, 0.5 * p + 0.5, 0.0)
    out_ref[...] = w * cut


_filt_call = pl.pallas_call(
    _filt_body,
    grid=(N_EDGES // EB,),
    in_specs=[
        pl.BlockSpec((EB, 1), lambda i: (i, 0)),
        pl.BlockSpec((NUM_GAUSSIANS, NUM_FILTERS), lambda i: (0, 0)),
        pl.BlockSpec((1, NUM_FILTERS), lambda i: (0, 0)),
        pl.BlockSpec((NUM_FILTERS, NUM_FILTERS), lambda i: (0, 0)),
        pl.BlockSpec((1, NUM_FILTERS), lambda i: (0, 0)),
    ],
    out_specs=pl.BlockSpec((EB, NUM_FILTERS), lambda i: (i, 0)),
    out_shape=jax.ShapeDtypeStruct((N_EDGES, NUM_FILTERS), jnp.float32),
)


def _scatter_body(in_hbm, src_hbm, dst_hbm, filt_hbm, out_hbm,
                  acc, srcv, dstv, filtv, inv, msgv, sem):
    c = lax.axis_index("c")
    s = lax.axis_index("s")
    wid = s * NC + c

    def zrow(i, _):
        for j in range(NUM_FILTERS // 16):
            msgv[i, pl.ds(j * 16, 16)] = jnp.zeros((16,), jnp.float32)
        return 0

    lax.fori_loop(0, ZCHUNK, zrow, 0)

    def zcopy(k, _):
        chunk = s + k * NS

        @pl.when(chunk < NCHUNK)
        def _():
            pltpu.sync_copy(msgv, acc.at[pl.ds(chunk * ZCHUNK, ZCHUNK)])

        return 0

    lax.fori_loop(0, (NCHUNK + NS - 1) // NS, zcopy, 0)
    plsc.subcore_barrier()

    def block(b, _):
        e0 = wid * EPW + b * B3
        pltpu.sync_copy(src_hbm.at[pl.ds(e0, B3)], srcv)
        pltpu.sync_copy(dst_hbm.at[pl.ds(e0, B3)], dstv)
        pltpu.sync_copy(filt_hbm.at[pl.ds(e0, B3)], filtv)
        pltpu.async_copy(in_hbm.at[srcv], inv, sem).wait()

        def row(i, _):
            for j in range(NUM_FILTERS // 16):
                sl = pl.ds(j * 16, 16)
                msgv[i, sl] = filtv[i, sl] * inv[i, sl]
            return 0

        lax.fori_loop(0, B3, row, 0)
        pltpu.sync_copy(msgv, acc.at[dstv], add=True)
        return 0

    lax.fori_loop(0, EPW // B3, block, 0)
    plsc.subcore_barrier()

    def dump(k, _):
        chunk = s + k * NS

        @pl.when(chunk < NCHUNK)
        def _():
            r0 = chunk * ZCHUNK
            pltpu.sync_copy(acc.at[pl.ds(r0, ZCHUNK)],
                            out_hbm.at[pl.ds(c * N_NODES + r0, ZCHUNK)])

        return 0

    lax.fori_loop(0, (NCHUNK + NS - 1) // NS, dump, 0)


def _sum_body(a_ref, b_ref, o_ref):
    o_ref[...] = a_ref[...] + b_ref[...]


_sum_call = pl.pallas_call(
    _sum_body,
    grid=(10,),
    in_specs=[
        pl.BlockSpec((N_NODES // 10, NUM_FILTERS), lambda i: (i, 0)),
        pl.BlockSpec((N_NODES // 10, NUM_FILTERS), lambda i: (i, 0)),
    ],
    out_specs=pl.BlockSpec((N_NODES // 10, NUM_FILTERS), lambda i: (i, 0)),
    out_shape=jax.ShapeDtypeStruct((N_NODES, NUM_FILTERS), jnp.float32),
)


@functools.lru_cache(maxsize=1)
def _sc_kernels():
    mesh = plsc.VectorSubcoreMesh(core_axis_name="c", subcore_axis_name="s",
                                  num_cores=NC, num_subcores=NS)
    d2_kernel = pl.kernel(
        _d2_body,
        out_type=jax.ShapeDtypeStruct((N_EDGES,), jnp.float32),
        mesh=mesh,
        compiler_params=pltpu.CompilerParams(needs_layout_passes=False),
        scratch_types=[
            pltpu.VMEM((3 * N_NODES,), jnp.float32),
            pltpu.VMEM((B1,), jnp.int32),
            pltpu.VMEM((B1,), jnp.int32),
            pltpu.VMEM((B1,), jnp.float32),
        ],
    )
    scatter_kernel = pl.kernel(
        _scatter_body,
        out_type=jax.ShapeDtypeStruct((NC * N_NODES, NUM_FILTERS),
                                      jnp.float32),
        mesh=mesh,
        compiler_params=pltpu.CompilerParams(needs_layout_passes=False),
        scratch_types=[
            pltpu.VMEM_SHARED((N_NODES, NUM_FILTERS), jnp.float32),
            pltpu.VMEM((B3,), jnp.int32),
            pltpu.VMEM((B3,), jnp.int32),
            pltpu.VMEM((B3, NUM_FILTERS), jnp.float32),
            pltpu.VMEM((B3, NUM_FILTERS), jnp.float32),
            pltpu.VMEM((B3, NUM_FILTERS), jnp.float32),
            pltpu.SemaphoreType.DMA,
        ],
    )
    return d2_kernel, scatter_kernel


def kernel(positions, input, edge_index, weights1, biases1, weights2, biases2):
    _d2_kernel, _scatter_kernel = _sc_kernels()
    pos_flat = positions.reshape(-1)
    src = edge_index[0]
    dst = edge_index[1]
    d2 = _d2_kernel(pos_flat, src, dst)
    filt = _filt_call(d2.reshape(N_EDGES, 1), weights1,
                      biases1.reshape(1, NUM_FILTERS), weights2,
                      biases2.reshape(1, NUM_FILTERS))
    parts = _scatter_kernel(input, src, dst, filt)
    return _sum_call(parts[:N_NODES], parts[N_NODES:])
